# baseline (speedup 1.0000x reference)
import functools

import jax
import jax.numpy as jnp
from jax.experimental import pallas as pl
from jax.experimental.pallas import tpu as pltpu

_N_BLOCK = 1000


def _hosvd_body(nh_ref, u0e_ref, u1e_ref, u2e_ref, u3e_ref,
                gblk_ref, sublk_ref, out_ref):
    nh = nh_ref[...].astype(jnp.bfloat16)
    h0 = nh[:, 0, :]
    h1 = nh[:, 1, :]
    h2 = nh[:, 2, :]
    h3 = nh[:, 3, :]
    dot = functools.partial(jnp.dot, preferred_element_type=jnp.float32)
    a1 = dot(h1, u1e_ref[...])
    a2 = dot(h2, u2e_ref[...])
    a3 = dot(h3, u3e_ref[...])
    c3 = (a1 * a2 * a3).astype(jnp.bfloat16)
    z = dot(c3, gblk_ref[...])
    a0 = dot(h0, u0e_ref[...])
    p = (a0 * z).astype(jnp.bfloat16)
    out_ref[...] = dot(p, sublk_ref[...])


def kernel(neighbour_h, U0, U1, U2, U3, G_i, G_o, G_u,
           Ui_output, Uo_output, Uu_output):
    n, d, h = neighbour_h.shape
    r = G_i.shape[0]
    r2, r3 = r * r, r * r * r

    def exp_cols(u, which):
        blocks = []
        for g in range(3):
            ug = u[:, r * g:r * g + r]
            if which == 'm':
                b = jnp.tile(ug, (1, r2))
            elif which == 'l':
                b = jnp.tile(jnp.repeat(ug, r, axis=1), (1, r))
            else:
                b = jnp.repeat(ug, r2, axis=1)
            blocks.append(b)
        return jnp.concatenate(blocks, axis=1)

    u1e = exp_cols(U1, 'm').astype(jnp.bfloat16)
    u2e = exp_cols(U2, 'l').astype(jnp.bfloat16)
    u3e = exp_cols(U3, 'k').astype(jnp.bfloat16)
    u0e = jnp.concatenate(
        [jnp.repeat(U0[:, r * g:r * g + r], r, axis=1) for g in range(3)],
        axis=1).astype(jnp.bfloat16)

    def gq(g):
        return jnp.transpose(g, (2, 3, 4, 0, 1)).reshape(r3, r2)

    gblk = jax.scipy.linalg.block_diag(
        gq(G_i), gq(G_o), gq(G_u)).astype(jnp.bfloat16)
    sublk = jax.scipy.linalg.block_diag(
        jnp.tile(Ui_output, (r, 1)),
        jnp.tile(Uo_output, (r, 1)),
        jnp.tile(Uu_output, (r, 1))).astype(jnp.bfloat16)

    bn = _N_BLOCK
    grid = (n // bn,)
    out = pl.pallas_call(
        _hosvd_body,
        grid=grid,
        in_specs=[
            pl.BlockSpec((bn, d, h), lambda i: (i, 0, 0)),
            pl.BlockSpec(u0e.shape, lambda i: (0, 0)),
            pl.BlockSpec(u1e.shape, lambda i: (0, 0)),
            pl.BlockSpec(u2e.shape, lambda i: (0, 0)),
            pl.BlockSpec(u3e.shape, lambda i: (0, 0)),
            pl.BlockSpec(gblk.shape, lambda i: (0, 0)),
            pl.BlockSpec(sublk.shape, lambda i: (0, 0)),
        ],
        out_specs=pl.BlockSpec((bn, 3 * h), lambda i: (i, 0)),
        out_shape=jax.ShapeDtypeStruct((n, 3 * h), jnp.float32),
        compiler_params=pltpu.CompilerParams(
            dimension_semantics=("arbitrary",)),
    )(neighbour_h, u0e, u1e, u2e, u3e, gblk, sublk)
    return out


# speedup vs baseline: 1.0559x; 1.0559x over previous
import functools

import jax
import jax.numpy as jnp
from jax.experimental import pallas as pl
from jax.experimental.pallas import tpu as pltpu

_N_BLOCK = 1000
_R = 8
_H = 256
_EW = 3 * _R ** 3


def _iota2(shape, dim):
    return jax.lax.broadcasted_iota(jnp.int32, shape, dim)


def _sel(rows, cols, src_of_col):
    q = _iota2((rows, cols), 0)
    c = _iota2((rows, cols), 1)
    return (q == src_of_col(c)).astype(jnp.bfloat16)


def _hosvd_body(nh_ref, u0_ref, u1_ref, u2_ref, u3_ref,
                g2i_ref, g2o_ref, g2u_ref, uoi_ref, uoo_ref, uou_ref,
                out_ref, u0e_s, u1e_s, u2e_s, u3e_s, gq_s, su_s):
    r, r2, r3 = _R, _R * _R, _R ** 3
    dot = functools.partial(jnp.dot, preferred_element_type=jnp.float32)

    @pl.when(pl.program_id(0) == 0)
    def _build_weights():
        e1 = _sel(3 * r, _EW, lambda c: r * (c // r3) + c % r)
        e2 = _sel(3 * r, _EW, lambda c: r * (c // r3) + (c // r) % r)
        e3 = _sel(3 * r, _EW, lambda c: r * (c // r3) + (c // r2) % r)
        e0 = _sel(3 * r, 3 * r2, lambda c: r * (c // r2) + (c % r2) // r)
        u1e_s[...] = dot(u1_ref[...].astype(jnp.bfloat16), e1).astype(jnp.bfloat16)
        u2e_s[...] = dot(u2_ref[...].astype(jnp.bfloat16), e2).astype(jnp.bfloat16)
        u3e_s[...] = dot(u3_ref[...].astype(jnp.bfloat16), e3).astype(jnp.bfloat16)
        u0e_s[...] = dot(u0_ref[...].astype(jnp.bfloat16), e0).astype(jnp.bfloat16)
        eye = (_iota2((r2, r2), 0) == _iota2((r2, r2), 1)).astype(jnp.bfloat16)
        gq_s[...] = jnp.zeros(gq_s.shape, gq_s.dtype)
        su_s[...] = jnp.zeros(su_s.shape, su_s.dtype)
        for g, (g2, uo) in enumerate([(g2i_ref, uoi_ref), (g2o_ref, uoo_ref),
                                      (g2u_ref, uou_ref)]):
            t = jax.lax.dot_general(
                g2[...].astype(jnp.bfloat16), eye,
                (((0,), (0,)), ((), ())),
                preferred_element_type=jnp.float32)
            gq_s[r3 * g:r3 * (g + 1), r2 * g:r2 * (g + 1)] = (
                t.astype(jnp.bfloat16))
            su_s[r2 * g:r2 * (g + 1), _H * g:_H * (g + 1)] = jnp.tile(
                uo[...], (r, 1)).astype(jnp.bfloat16)

    nh = nh_ref[...].astype(jnp.bfloat16)
    a1 = dot(nh[:, 1, :], u1e_s[...])
    a2 = dot(nh[:, 2, :], u2e_s[...])
    a3 = dot(nh[:, 3, :], u3e_s[...])
    c3 = (a1 * a2 * a3).astype(jnp.bfloat16)
    z = dot(c3, gq_s[...])
    a0 = dot(nh[:, 0, :], u0e_s[...])
    p = (a0 * z).astype(jnp.bfloat16)
    out_ref[...] = dot(p, su_s[...])


def kernel(neighbour_h, U0, U1, U2, U3, G_i, G_o, G_u,
           Ui_output, Uo_output, Uu_output):
    n, d, h = neighbour_h.shape
    r = G_i.shape[0]
    r2, r3 = r * r, r ** 3
    g2i = G_i.reshape(r2, r3)
    g2o = G_o.reshape(r2, r3)
    g2u = G_u.reshape(r2, r3)

    bn = _N_BLOCK
    full = lambda a: pl.BlockSpec(a.shape, lambda i: (0,) * a.ndim)
    bf16 = jnp.bfloat16
    out = pl.pallas_call(
        _hosvd_body,
        grid=(n // bn,),
        in_specs=[pl.BlockSpec((bn, d, h), lambda i: (i, 0, 0)),
                  full(U0), full(U1), full(U2), full(U3),
                  full(g2i), full(g2o), full(g2u),
                  full(Ui_output), full(Uo_output), full(Uu_output)],
        out_specs=pl.BlockSpec((bn, 3 * h), lambda i: (i, 0)),
        out_shape=jax.ShapeDtypeStruct((n, 3 * h), jnp.float32),
        scratch_shapes=[pltpu.VMEM((h, 3 * r2), bf16),
                        pltpu.VMEM((h, _EW), bf16),
                        pltpu.VMEM((h, _EW), bf16),
                        pltpu.VMEM((h, _EW), bf16),
                        pltpu.VMEM((3 * r3, 3 * r2), bf16),
                        pltpu.VMEM((3 * r2, 3 * h), bf16)],
        compiler_params=pltpu.CompilerParams(
            dimension_semantics=("arbitrary",)),
    )(neighbour_h, U0, U1, U2, U3, g2i, g2o, g2u,
      Ui_output, Uo_output, Uu_output)
    return out


# speedup vs baseline: 1.0742x; 1.0174x over previous
import functools

import jax
import jax.numpy as jnp
from jax.experimental import pallas as pl
from jax.experimental.pallas import tpu as pltpu

_N_BLOCK = 1000
_R = 8
_H = 256
_EW = 3 * _R ** 3


def _iota2(shape, dim):
    return jax.lax.broadcasted_iota(jnp.int32, shape, dim)


def _sel(rows, cols, src_of_col):
    q = _iota2((rows, cols), 0)
    c = _iota2((rows, cols), 1)
    return (q == src_of_col(c)).astype(jnp.bfloat16)


def _hosvd_body(nh_ref, u0_ref, u1_ref, u2_ref, u3_ref,
                gall_ref, uoi_ref, uoo_ref, uou_ref,
                out_ref, u0e_s, u1e_s, u2e_s, u3e_s, gq_s, su_s):
    r, r2, r3 = _R, _R * _R, _R ** 3
    dot = functools.partial(jnp.dot, preferred_element_type=jnp.float32)

    @pl.when(pl.program_id(0) == 0)
    def _build_weights():
        e1 = _sel(3 * r, _EW, lambda c: r * (c // r3) + c % r)
        e2 = _sel(3 * r, _EW, lambda c: r * (c // r3) + (c // r) % r)
        e3 = _sel(3 * r, _EW, lambda c: r * (c // r3) + (c // r2) % r)
        e0 = _sel(3 * r, 3 * r2, lambda c: r * (c // r2) + (c % r2) // r)
        u1e_s[...] = dot(u1_ref[...].astype(jnp.bfloat16), e1).astype(jnp.bfloat16)
        u2e_s[...] = dot(u2_ref[...].astype(jnp.bfloat16), e2).astype(jnp.bfloat16)
        u3e_s[...] = dot(u3_ref[...].astype(jnp.bfloat16), e3).astype(jnp.bfloat16)
        u0e_s[...] = dot(u0_ref[...].astype(jnp.bfloat16), e0).astype(jnp.bfloat16)
        eye = (_iota2((r2, r2), 0) == _iota2((r2, r2), 1)).astype(jnp.bfloat16)
        gq_s[...] = jnp.zeros(gq_s.shape, gq_s.dtype)
        su_s[...] = jnp.zeros(su_s.shape, su_s.dtype)
        for g, uo in enumerate([uoi_ref, uoo_ref, uou_ref]):
            t = jax.lax.dot_general(
                gall_ref[r2 * g:r2 * (g + 1), :].astype(jnp.bfloat16), eye,
                (((0,), (0,)), ((), ())),
                preferred_element_type=jnp.float32)
            gq_s[r3 * g:r3 * (g + 1), r2 * g:r2 * (g + 1)] = (
                t.astype(jnp.bfloat16))
            su_s[r2 * g:r2 * (g + 1), _H * g:_H * (g + 1)] = jnp.tile(
                uo[...], (r, 1)).astype(jnp.bfloat16)

    nh = nh_ref[...].astype(jnp.bfloat16)
    a1 = dot(nh[:, 1, :], u1e_s[...])
    a2 = dot(nh[:, 2, :], u2e_s[...])
    a3 = dot(nh[:, 3, :], u3e_s[...])
    c3 = (a1 * a2 * a3).astype(jnp.bfloat16)
    z = dot(c3, gq_s[...])
    a0 = dot(nh[:, 0, :], u0e_s[...])
    p = (a0 * z).astype(jnp.bfloat16)
    out_ref[...] = dot(p, su_s[...])


def kernel(neighbour_h, U0, U1, U2, U3, G_i, G_o, G_u,
           Ui_output, Uo_output, Uu_output):
    n, d, h = neighbour_h.shape
    r = G_i.shape[0]
    r2, r3 = r * r, r ** 3
    gall = jnp.concatenate([G_i.reshape(r2, r3), G_o.reshape(r2, r3),
                            G_u.reshape(r2, r3)], axis=0)

    bn = _N_BLOCK
    full = lambda a: pl.BlockSpec(a.shape, lambda i: (0,) * a.ndim)
    bf16 = jnp.bfloat16
    out = pl.pallas_call(
        _hosvd_body,
        grid=(n // bn,),
        in_specs=[pl.BlockSpec((bn, d, h), lambda i: (i, 0, 0)),
                  full(U0), full(U1), full(U2), full(U3),
                  full(gall),
                  full(Ui_output), full(Uo_output), full(Uu_output)],
        out_specs=pl.BlockSpec((bn, 3 * h), lambda i: (i, 0)),
        out_shape=jax.ShapeDtypeStruct((n, 3 * h), jnp.float32),
        scratch_shapes=[pltpu.VMEM((h, 3 * r2), bf16),
                        pltpu.VMEM((h, _EW), bf16),
                        pltpu.VMEM((h, _EW), bf16),
                        pltpu.VMEM((h, _EW), bf16),
                        pltpu.VMEM((3 * r3, 3 * r2), bf16),
                        pltpu.VMEM((3 * r2, 3 * h), bf16)],
        compiler_params=pltpu.CompilerParams(
            dimension_semantics=("arbitrary",)),
    )(neighbour_h, U0, U1, U2, U3, gall,
      Ui_output, Uo_output, Uu_output)
    return out
